# Initial kernel scaffold; baseline (speedup 1.0000x reference)
import jax
import jax.numpy as jnp
from jax.experimental import pallas as pl


def kernel(x):
    raise NotImplementedError("write your pallas kernel here")


# speedup vs baseline: 4.1988x; 4.1988x over previous
import numpy as np
import jax
import jax.numpy as jnp
from jax.experimental import pallas as pl
from jax.experimental.pallas import tpu as pltpu

_NBINS = 100
_P = 0.95
_BIG = 1e30


def _radial_consts(h, w):
    cy, cx = h // 2, w // 2
    yy = jnp.arange(h, dtype=jnp.float32) - cy
    xx = jnp.arange(w, dtype=jnp.float32) - cx
    md = ((h // 2) ** 2 + (w // 2) ** 2) ** 0.5
    yy = yy / (md + 1e-06)
    xx = xx / (md + 1e-06)
    gy, gx = jnp.meshgrid(yy, xx, indexing="ij")
    rr = jnp.sqrt(gx * gx + gy * gy)
    r_max = jnp.max(rr)
    bin_idx = jnp.floor(rr / r_max * _NBINS).astype(jnp.int32)
    bidx = jnp.where(bin_idx < _NBINS,
                     jnp.clip(bin_idx, 0, _NBINS - 1), 127).astype(jnp.float32)
    edges = jnp.linspace(0.0, r_max, _NBINS + 1)
    radii = (edges[:-1] + edges[1:]) * 0.5
    radii_pad = jnp.concatenate(
        [radii, jnp.full((28,), radii[-1], jnp.float32)]).reshape(1, 1, 128)
    return rr, bidx, radii_pad


def _dft_mats(n):
    k = np.arange(n, dtype=np.float64) - n // 2
    y = np.arange(n, dtype=np.float64)
    th = (-2.0 * np.pi / n) * np.outer(k, y)
    return (jnp.asarray(np.cos(th), jnp.float32),
            jnp.asarray(np.sin(th), jnp.float32))


def _make_body(C, H, W):
    HIGH = jax.lax.Precision.HIGHEST

    def mm(a, b, dims):
        return jax.lax.dot_general(a, b, (dims, ((), ())),
                                   precision=HIGH,
                                   preferred_element_type=jnp.float32)

    def body(x_ref, rr_ref, bidx_ref, radii_ref, dr_ref, di_ref,
             ih_ref, il_ref, d0_ref, ml_ref, mh_ref,
             fr_scr, fi_scr, pw_scr):
        s = pl.program_id(1)
        X = x_ref[0, 0]
        Rr = dr_ref[...]
        Ri = di_ref[...]

        @pl.when(s < C)
        def _fwd():
            Yr = mm(X, Rr, ((1,), (1,)))
            Yi = mm(X, Ri, ((1,), (1,)))
            Fr = mm(Rr, Yr, ((1,), (0,))) - mm(Ri, Yi, ((1,), (0,)))
            Fi = mm(Rr, Yi, ((1,), (0,))) + mm(Ri, Yr, ((1,), (0,)))
            fr_scr[pl.ds(s, 1)] = Fr[None]
            fi_scr[pl.ds(s, 1)] = Fi[None]
            p = Fr * Fr + Fi * Fi

            @pl.when(s == 0)
            def _():
                pw_scr[...] = p

            @pl.when(s > 0)
            def _():
                pw_scr[...] = pw_scr[...] + p

        @pl.when(s == C - 1)
        def _d0():
            pw = pw_scr[...]
            total = jnp.maximum(jnp.sum(pw), 1e-12)
            bidx = bidx_ref[...]
            radii_row = radii_ref[0]
            lane = jax.lax.broadcasted_iota(
                jnp.int32, (1, 128), 1).astype(jnp.float32)
            thr = _P * total

            def loop(k, carry):
                cum, d0 = carry
                kf = k.astype(jnp.float32)
                sk = jnp.sum(jnp.where(bidx == kf, pw, 0.0))
                cum = cum + sk
                rk = jnp.sum(jnp.where(lane == kf, radii_row, 0.0))
                d0 = jnp.minimum(d0, jnp.where(cum >= thr, rk, _BIG))
                return cum, d0

            _, d0 = jax.lax.fori_loop(
                0, _NBINS, loop,
                (jnp.float32(0.0), jnp.float32(_BIG)))
            d0 = jnp.where(d0 > 0.5 * _BIG, radii_ref[0, 0, _NBINS - 1], d0)
            d0_ref[...] = jnp.broadcast_to(d0, (1, 1, 1))

        @pl.when(s >= C)
        def _inv():
            cc = s - C
            Fr = fr_scr[pl.ds(cc, 1)][0]
            Fi = fi_scr[pl.ds(cc, 1)][0]
            d0 = d0_ref[...][0, 0, 0]
            m = (rr_ref[...] <= d0).astype(jnp.float32)
            Gr = Fr * m
            Gi = Fi * m
            Zr = mm(Rr, Gr, ((0,), (0,))) + mm(Ri, Gi, ((0,), (0,)))
            Zi = mm(Rr, Gi, ((0,), (0,))) - mm(Ri, Gr, ((0,), (0,)))
            IL = (mm(Zr, Rr, ((1,), (0,))) +
                  mm(Zi, Ri, ((1,), (0,)))) * (1.0 / (H * W))
            il_ref[0, 0] = IL
            ih_ref[0, 0] = X - IL
            ml_ref[0, 0] = m
            mh_ref[0, 0] = 1.0 - m

    return body


def kernel(x):
    B, C, H, W = x.shape
    rr, bidx, radii_pad = _radial_consts(H, W)
    Dr, Di = _dft_mats(H)

    body = _make_body(C, H, W)

    def xmap(b, s):
        return (b, jnp.where(s < C, s, s - C), 0, 0)

    def omap(b, s):
        return (b, jnp.where(s < C, 0, s - C), 0, 0)

    outs = pl.pallas_call(
        body,
        grid=(B, 2 * C),
        in_specs=[
            pl.BlockSpec((1, 1, H, W), xmap),
            pl.BlockSpec((H, W), lambda b, s: (0, 0)),
            pl.BlockSpec((H, W), lambda b, s: (0, 0)),
            pl.BlockSpec((1, 1, 128), lambda b, s: (0, 0, 0)),
            pl.BlockSpec((H, H), lambda b, s: (0, 0)),
            pl.BlockSpec((H, H), lambda b, s: (0, 0)),
        ],
        out_specs=[
            pl.BlockSpec((1, 1, H, W), omap),
            pl.BlockSpec((1, 1, H, W), omap),
            pl.BlockSpec((1, 1, 1), lambda b, s: (b, 0, 0)),
            pl.BlockSpec((1, 1, H, W), lambda b, s: (b, 0, 0, 0)),
            pl.BlockSpec((1, 1, H, W), lambda b, s: (b, 0, 0, 0)),
        ],
        out_shape=[
            jax.ShapeDtypeStruct((B, C, H, W), jnp.float32),
            jax.ShapeDtypeStruct((B, C, H, W), jnp.float32),
            jax.ShapeDtypeStruct((B, 1, 1), jnp.float32),
            jax.ShapeDtypeStruct((B, 1, H, W), jnp.float32),
            jax.ShapeDtypeStruct((B, 1, H, W), jnp.float32),
        ],
        scratch_shapes=[
            pltpu.VMEM((C, H, W), jnp.float32),
            pltpu.VMEM((C, H, W), jnp.float32),
            pltpu.VMEM((H, W), jnp.float32),
        ],
        compiler_params=pltpu.CompilerParams(
            dimension_semantics=("arbitrary", "arbitrary")),
    )(x, rr, bidx, radii_pad, Dr, Di)

    i_high, i_low, d0, mask_low, mask_high = outs
    return i_high, i_low, d0.reshape(B), mask_low, mask_high


# speedup vs baseline: 6.2147x; 1.4801x over previous
import numpy as np
import jax
import jax.numpy as jnp
from jax.experimental import pallas as pl
from jax.experimental.pallas import tpu as pltpu

_NBINS = 100
_P = 0.95
_BIG = 1e30


def _radial_consts(h, w):
    cy, cx = h // 2, w // 2
    yy = jnp.arange(h, dtype=jnp.float32) - cy
    xx = jnp.arange(w, dtype=jnp.float32) - cx
    md = ((h // 2) ** 2 + (w // 2) ** 2) ** 0.5
    yy = yy / (md + 1e-06)
    xx = xx / (md + 1e-06)
    gy, gx = jnp.meshgrid(yy, xx, indexing="ij")
    rr = jnp.sqrt(gx * gx + gy * gy)
    r_max = jnp.max(rr)
    bin_idx = jnp.floor(rr / r_max * _NBINS).astype(jnp.int32)
    bidx = jnp.where(bin_idx < _NBINS,
                     jnp.clip(bin_idx, 0, _NBINS - 1), 127).astype(jnp.float32)
    edges = jnp.linspace(0.0, r_max, _NBINS + 1)
    radii = (edges[:-1] + edges[1:]) * 0.5
    radii_pad = jnp.concatenate(
        [radii, jnp.full((28,), radii[-1], jnp.float32)]).reshape(1, 1, 128)
    return rr, bidx, radii_pad


def _dft_mats(n):
    k = np.arange(n, dtype=np.float64) - n // 2
    y = np.arange(n, dtype=np.float64)
    th = (-2.0 * np.pi / n) * np.outer(k, y)
    return (jnp.asarray(np.cos(th), jnp.float32),
            jnp.asarray(np.sin(th), jnp.float32))


def _split_hi_lo(a):
    hi = a.astype(jnp.bfloat16)
    lo = (a - hi.astype(jnp.float32)).astype(jnp.bfloat16)
    return hi, lo


def _make_body(C, H, W):
    def dotg(a, b, dims):
        return jax.lax.dot_general(a, b, (dims, ((), ())),
                                   preferred_element_type=jnp.float32)

    def mmx(ahl, bhl, dims):
        ah, al = ahl
        bh, bl = bhl
        return (dotg(ah, bh, dims) + dotg(ah, bl, dims)
                + dotg(al, bh, dims))

    def body(x_ref, rr_ref, bidx_ref, radii_ref,
             rrh_ref, rrl_ref, rih_ref, ril_ref,
             rsh_ref, rsl_ref, rdh_ref, rdl_ref,
             ih_ref, il_ref, d0_ref, ml_ref, mh_ref,
             fr_scr, fi_scr, pw_scr):
        s = pl.program_id(1)
        X = x_ref[0, 0]
        Rr = (rrh_ref[...], rrl_ref[...])
        Ri = (rih_ref[...], ril_ref[...])
        Rs = (rsh_ref[...], rsl_ref[...])
        Rd = (rdh_ref[...], rdl_ref[...])

        @pl.when(s < C)
        def _fwd():
            Xs = _split_hi_lo(X)
            Yr = mmx(Xs, Rr, ((1,), (1,)))
            Yi = mmx(Xs, Ri, ((1,), (1,)))
            P1 = mmx(Rr, _split_hi_lo(Yr), ((1,), (0,)))
            P2 = mmx(Ri, _split_hi_lo(Yi), ((1,), (0,)))
            P3 = mmx(Rs, _split_hi_lo(Yr + Yi), ((1,), (0,)))
            Fr = P1 - P2
            Fi = P3 - P1 - P2
            fr_scr[pl.ds(s, 1)] = Fr[None]
            fi_scr[pl.ds(s, 1)] = Fi[None]
            p = Fr * Fr + Fi * Fi

            @pl.when(s == 0)
            def _():
                pw_scr[...] = p

            @pl.when(s > 0)
            def _():
                pw_scr[...] = pw_scr[...] + p

        @pl.when(s == C - 1)
        def _d0():
            pw = pw_scr[...]
            total = jnp.maximum(jnp.sum(pw), 1e-12)
            bidx = bidx_ref[...]
            radii_row = radii_ref[0]
            lane = jax.lax.broadcasted_iota(
                jnp.int32, (1, 128), 1).astype(jnp.float32)
            thr = _P * total

            def loop(k, carry):
                cum, d0 = carry
                kf = k.astype(jnp.float32)
                sk = jnp.sum(jnp.where(bidx == kf, pw, 0.0))
                cum = cum + sk
                rk = jnp.sum(jnp.where(lane == kf, radii_row, 0.0))
                d0 = jnp.minimum(d0, jnp.where(cum >= thr, rk, _BIG))
                return cum, d0

            _, d0 = jax.lax.fori_loop(
                0, _NBINS, loop,
                (jnp.float32(0.0), jnp.float32(_BIG)))
            d0 = jnp.where(d0 > 0.5 * _BIG, radii_ref[0, 0, _NBINS - 1], d0)
            d0_ref[...] = jnp.broadcast_to(d0, (1, 1, 1))

        @pl.when(s >= C)
        def _inv():
            cc = s - C
            Fr = fr_scr[pl.ds(cc, 1)][0]
            Fi = fi_scr[pl.ds(cc, 1)][0]
            d0 = d0_ref[...][0, 0, 0]
            m = (rr_ref[...] <= d0).astype(jnp.float32)
            Gr = Fr * m
            Gi = Fi * m
            P1 = mmx(Rr, _split_hi_lo(Gr), ((0,), (0,)))
            M2 = mmx(Ri, _split_hi_lo(Gi), ((0,), (0,)))
            P3 = mmx(Rd, _split_hi_lo(Gr + Gi), ((0,), (0,)))
            Zr = P1 + M2
            Zi = P3 - P1 + M2
            IL = (mmx(_split_hi_lo(Zr), Rr, ((1,), (0,))) +
                  mmx(_split_hi_lo(Zi), Ri, ((1,), (0,)))) * (1.0 / (H * W))
            il_ref[0, 0] = IL
            ih_ref[0, 0] = X - IL
            ml_ref[0, 0] = m
            mh_ref[0, 0] = 1.0 - m

    return body


def kernel(x):
    B, C, H, W = x.shape
    rr, bidx, radii_pad = _radial_consts(H, W)
    Dr, Di = _dft_mats(H)
    mats = []
    for mat in (Dr, Di, Dr + Di, Dr - Di):
        mats.extend(_split_hi_lo(mat))

    body = _make_body(C, H, W)

    def xmap(b, s):
        return (b, jnp.where(s < C, s, s - C), 0, 0)

    def omap(b, s):
        return (b, jnp.where(s < C, 0, s - C), 0, 0)

    full2d = pl.BlockSpec((H, H), lambda b, s: (0, 0))

    outs = pl.pallas_call(
        body,
        grid=(B, 2 * C),
        in_specs=[
            pl.BlockSpec((1, 1, H, W), xmap),
            pl.BlockSpec((H, W), lambda b, s: (0, 0)),
            pl.BlockSpec((H, W), lambda b, s: (0, 0)),
            pl.BlockSpec((1, 1, 128), lambda b, s: (0, 0, 0)),
        ] + [full2d] * 8,
        out_specs=[
            pl.BlockSpec((1, 1, H, W), omap),
            pl.BlockSpec((1, 1, H, W), omap),
            pl.BlockSpec((1, 1, 1), lambda b, s: (b, 0, 0)),
            pl.BlockSpec((1, 1, H, W), lambda b, s: (b, 0, 0, 0)),
            pl.BlockSpec((1, 1, H, W), lambda b, s: (b, 0, 0, 0)),
        ],
        out_shape=[
            jax.ShapeDtypeStruct((B, C, H, W), jnp.float32),
            jax.ShapeDtypeStruct((B, C, H, W), jnp.float32),
            jax.ShapeDtypeStruct((B, 1, 1), jnp.float32),
            jax.ShapeDtypeStruct((B, 1, H, W), jnp.float32),
            jax.ShapeDtypeStruct((B, 1, H, W), jnp.float32),
        ],
        scratch_shapes=[
            pltpu.VMEM((C, H, W), jnp.float32),
            pltpu.VMEM((C, H, W), jnp.float32),
            pltpu.VMEM((H, W), jnp.float32),
        ],
        compiler_params=pltpu.CompilerParams(
            dimension_semantics=("arbitrary", "arbitrary")),
    )(x, rr, bidx, radii_pad, *mats)

    i_high, i_low, d0, mask_low, mask_high = outs
    return i_high, i_low, d0.reshape(B), mask_low, mask_high


# speedup vs baseline: 13.0714x; 2.1033x over previous
import numpy as np
import jax
import jax.numpy as jnp
from jax.experimental import pallas as pl
from jax.experimental.pallas import tpu as pltpu

_NBINS = 100
_P = 0.95


def _radial_consts(h, w):
    cy, cx = h // 2, w // 2
    yy = jnp.arange(h, dtype=jnp.float32) - cy
    xx = jnp.arange(w, dtype=jnp.float32) - cx
    md = ((h // 2) ** 2 + (w // 2) ** 2) ** 0.5
    yy = yy / (md + 1e-06)
    xx = xx / (md + 1e-06)
    gy, gx = jnp.meshgrid(yy, xx, indexing="ij")
    rr = jnp.sqrt(gx * gx + gy * gy)
    r_max = jnp.max(rr)
    bin_idx = jnp.floor(rr / r_max * _NBINS).astype(jnp.int32)
    bidx = jnp.where(bin_idx < _NBINS,
                     jnp.clip(bin_idx, 0, _NBINS - 1), 127).astype(jnp.float32)
    edges = jnp.linspace(0.0, r_max, _NBINS + 1)
    radii = (edges[:-1] + edges[1:]) * 0.5
    radii_pad = jnp.concatenate(
        [radii, jnp.full((28,), radii[-1], jnp.float32)]).reshape(1, 1, 128)
    return rr, bidx, radii_pad


def _dft_mats(n):
    k = np.arange(n, dtype=np.float64) - n // 2
    y = np.arange(n, dtype=np.float64)
    th = (-2.0 * np.pi / n) * np.outer(k, y)
    return (jnp.asarray(np.cos(th), jnp.float32),
            jnp.asarray(np.sin(th), jnp.float32))


def _split_hi_lo(a):
    hi = a.astype(jnp.bfloat16)
    lo = (a - hi.astype(jnp.float32)).astype(jnp.bfloat16)
    return hi, lo


def _make_body(C, H, W):
    def dotg(a, b, dims):
        return jax.lax.dot_general(a, b, (dims, ((), ())),
                                   preferred_element_type=jnp.float32)

    def dcat1(v):
        vh, vl = _split_hi_lo(v)
        return jnp.concatenate([vh, vh, vl], axis=1)

    def dcat0(v):
        vh, vl = _split_hi_lo(v)
        return jnp.concatenate([vh, vl, vh], axis=0)

    def body(x_ref, rr_ref, bidx_ref, radii_ref,
             f1r_ref, f1i_ref, f2r_ref, f2i_ref, f2s_ref,
             i1r_ref, i1i_ref, i1d_ref, i2r_ref, i2i_ref,
             ih_ref, il_ref, d0_ref, ml_ref, mh_ref,
             fr_scr, fi_scr, pw_scr):
        s = pl.program_id(1)
        X = x_ref[0, 0]

        @pl.when(s < C)
        def _fwd():
            Xc = dcat1(X)
            Yr = dotg(Xc, f1r_ref[...], ((1,), (1,)))
            Yi = dotg(Xc, f1i_ref[...], ((1,), (1,)))
            P1 = dotg(f2r_ref[...], dcat0(Yr), ((1,), (0,)))
            P2 = dotg(f2i_ref[...], dcat0(Yi), ((1,), (0,)))
            P3 = dotg(f2s_ref[...], dcat0(Yr + Yi), ((1,), (0,)))
            Fr = P1 - P2
            Fi = P3 - P1 - P2
            fr_scr[pl.ds(s, 1)] = Fr[None]
            fi_scr[pl.ds(s, 1)] = Fi[None]
            p = Fr * Fr + Fi * Fi

            @pl.when(s == 0)
            def _():
                pw_scr[...] = p

            @pl.when(s > 0)
            def _():
                pw_scr[...] = pw_scr[...] + p

        @pl.when(s == C - 1)
        def _d0():
            pw = pw_scr[...]
            total = jnp.maximum(jnp.sum(pw), 1e-12)
            bidx = bidx_ref[...]
            thr = _P * total

            def loop(_, carry):
                lo, hi = carry
                mid = (lo + hi) // 2
                c = jnp.sum(jnp.where(bidx <= mid.astype(jnp.float32),
                                      pw, 0.0))
                take = c >= thr
                return (jnp.where(take, lo, mid), jnp.where(take, mid, hi))

            lo, hi = jax.lax.fori_loop(
                0, 7, loop, (jnp.int32(-1), jnp.int32(_NBINS)))
            kf = jnp.minimum(hi, _NBINS - 1).astype(jnp.float32)
            lane = jax.lax.broadcasted_iota(
                jnp.int32, (1, 128), 1).astype(jnp.float32)
            d0 = jnp.sum(jnp.where(lane == kf, radii_ref[0], 0.0))
            d0_ref[...] = jnp.broadcast_to(d0, (1, 1, 1))

        @pl.when(s >= C)
        def _inv():
            cc = s - C
            Fr = fr_scr[pl.ds(cc, 1)][0]
            Fi = fi_scr[pl.ds(cc, 1)][0]
            d0 = d0_ref[...][0, 0, 0]
            m = (rr_ref[...] <= d0).astype(jnp.float32)
            Gr = Fr * m
            Gi = Fi * m
            P1 = dotg(i1r_ref[...], dcat0(Gr), ((0,), (0,)))
            M2 = dotg(i1i_ref[...], dcat0(Gi), ((0,), (0,)))
            P3 = dotg(i1d_ref[...], dcat0(Gr + Gi), ((0,), (0,)))
            Zr = P1 + M2
            Zi = P3 - P1 + M2
            IL = (dotg(dcat1(Zr), i2r_ref[...], ((1,), (0,))) +
                  dotg(dcat1(Zi), i2i_ref[...], ((1,), (0,)))) * (1.0 / (H * W))
            il_ref[0, 0] = IL
            ih_ref[0, 0] = X - IL
            ml_ref[0, 0] = m
            mh_ref[0, 0] = 1.0 - m

    return body


def kernel(x):
    B, C, H, W = x.shape
    rr, bidx, radii_pad = _radial_consts(H, W)
    Dr, Di = _dft_mats(H)

    Rrh, Rrl = _split_hi_lo(Dr)
    Rih, Ril = _split_hi_lo(Di)
    Rsh, Rsl = _split_hi_lo(Dr + Di)
    Rdh, Rdl = _split_hi_lo(Dr - Di)
    cat = jnp.concatenate
    mats = [
        cat([Rrh, Rrl, Rrh], axis=1),
        cat([Rih, Ril, Rih], axis=1),
        cat([Rrh, Rrh, Rrl], axis=1),
        cat([Rih, Rih, Ril], axis=1),
        cat([Rsh, Rsh, Rsl], axis=1),
        cat([Rrh, Rrh, Rrl], axis=0),
        cat([Rih, Rih, Ril], axis=0),
        cat([Rdh, Rdh, Rdl], axis=0),
        cat([Rrh, Rrl, Rrh], axis=0),
        cat([Rih, Ril, Rih], axis=0),
    ]

    body = _make_body(C, H, W)

    def xmap(b, s):
        return (b, jnp.where(s < C, s, s - C), 0, 0)

    def omap(b, s):
        return (b, jnp.where(s < C, 0, s - C), 0, 0)

    wide = pl.BlockSpec((H, 3 * H), lambda b, s: (0, 0))
    tall = pl.BlockSpec((3 * H, H), lambda b, s: (0, 0))

    outs = pl.pallas_call(
        body,
        grid=(B, 2 * C),
        in_specs=[
            pl.BlockSpec((1, 1, H, W), xmap),
            pl.BlockSpec((H, W), lambda b, s: (0, 0)),
            pl.BlockSpec((H, W), lambda b, s: (0, 0)),
            pl.BlockSpec((1, 1, 128), lambda b, s: (0, 0, 0)),
        ] + [wide] * 5 + [tall] * 5,
        out_specs=[
            pl.BlockSpec((1, 1, H, W), omap),
            pl.BlockSpec((1, 1, H, W), omap),
            pl.BlockSpec((1, 1, 1), lambda b, s: (b, 0, 0)),
            pl.BlockSpec((1, 1, H, W), lambda b, s: (b, 0, 0, 0)),
            pl.BlockSpec((1, 1, H, W), lambda b, s: (b, 0, 0, 0)),
        ],
        out_shape=[
            jax.ShapeDtypeStruct((B, C, H, W), jnp.float32),
            jax.ShapeDtypeStruct((B, C, H, W), jnp.float32),
            jax.ShapeDtypeStruct((B, 1, 1), jnp.float32),
            jax.ShapeDtypeStruct((B, 1, H, W), jnp.float32),
            jax.ShapeDtypeStruct((B, 1, H, W), jnp.float32),
        ],
        scratch_shapes=[
            pltpu.VMEM((C, H, W), jnp.float32),
            pltpu.VMEM((C, H, W), jnp.float32),
            pltpu.VMEM((H, W), jnp.float32),
        ],
        compiler_params=pltpu.CompilerParams(
            dimension_semantics=("arbitrary", "arbitrary")),
    )(x, rr, bidx, radii_pad, *mats)

    i_high, i_low, d0, mask_low, mask_high = outs
    return i_high, i_low, d0.reshape(B), mask_low, mask_high
